# Initial kernel scaffold; baseline (speedup 1.0000x reference)
import jax
import jax.numpy as jnp
from jax.experimental import pallas as pl


def kernel(images, anc_bases):
    raise NotImplementedError("write your pallas kernel here")


# speedup vs baseline: 418.5800x; 418.5800x over previous
import functools

import jax
import jax.numpy as jnp
from jax import lax
from jax.experimental import pallas as pl
from jax.experimental.pallas import tpu as pltpu
from jax.experimental.pallas import tpu_sc as plsc

B, C, H, W = 8, 3, 512, 512
N = 600
GH = GW = 31
NG = B * N
GPG = 8
NGROUPS = NG // GPG
NTILES = 32
ROWS = C * GH
ROWS_PAD = 96
CHUNK = 64
TW = 640
TCHUNKS = TW // CHUNK
TH = 544
GSZ = C * GH * GW
OUT_STG = GPG * GSZ
NROWS_TBL = B * C * 2 * TH * TCHUNKS

_MAX_K = (NGROUPS + NTILES - 1) // NTILES


def _table_body(img_ref, out_ref):
    img = img_ref[0]
    p = jnp.pad(img, ((1, 1), (1, 1)))
    a = ((p[0:513, 0:513] + p[0:513, 1:514]) + p[1:514, 0:513]) + p[1:514, 1:514]
    s = 0.25 * a
    out_ref[0, 0] = jnp.pad(s, ((15, 16), (15, 112)))
    out_ref[0, 1] = jnp.pad(s, ((15, 16), (47, 80)))


def _build_table(images):
    imgs = images.reshape(B * C, H, W)
    tab = pl.pallas_call(
        _table_body,
        grid=(B * C,),
        in_specs=[pl.BlockSpec((1, H, W), lambda i: (i, 0, 0))],
        out_specs=pl.BlockSpec((1, 2, TH, TW), lambda i: (i, 0, 0, 0)),
        out_shape=jax.ShapeDtypeStruct((B * C, 2, TH, TW), jnp.float32),
    )(imgs)
    return tab.reshape(NROWS_TBL, CHUNK)


def _sc_body(table_hbm, params_hbm, out_hbm, pre_v, idx_v, params_v, stage_v,
             outstage_v, sem_g, sem_p, sem_o):
    wid = lax.axis_index("s") * 2 + lax.axis_index("c")

    for t in range(ROWS_PAD // 16):
        r16 = lax.iota(jnp.int32, 16) + (16 * t)
        c = 2 + ((r16 - GH) >> 31) + ((r16 - 2 * GH) >> 31)
        i = r16 - c * GH
        pre_v[pl.ds(16 * t, 16)] = c * (2 * TH * TCHUNKS) + i * TCHUNKS

    def group_body(k, _):
        gid = wid + NTILES * k

        @pl.when(gid < NGROUPS)
        def _():
            g0 = gid * GPG
            pltpu.async_copy(params_hbm.at[pl.ds(g0, GPG)], params_v, sem_p).wait()

            def glimpse_body(j, _):
                pv = params_v[j]
                sp = pv[0]
                p = pv[1]
                for t in range(ROWS_PAD // 16):
                    idx_v[pl.ds(16 * t, 16)] = pre_v[pl.ds(16 * t, 16)] + sp
                pltpu.async_copy(table_hbm.at[idx_v], stage_v, sem_g).wait()

                def row_body(r, _):
                    o = j * GSZ + r * GW
                    v0 = stage_v[r, pl.ds(p, 16)]
                    v1 = stage_v[r, pl.ds(p + 15, 16)]
                    outstage_v[pl.ds(o, 16)] = v0
                    outstage_v[pl.ds(o + 15, 16)] = v1
                    return 0

                lax.fori_loop(0, ROWS, row_body, 0)
                return 0

            lax.fori_loop(0, GPG, glimpse_body, 0)
            pltpu.async_copy(
                outstage_v, out_hbm.at[pl.ds(g0 * GSZ, OUT_STG)], sem_o
            ).wait()

        return 0

    lax.fori_loop(0, _MAX_K, group_body, 0)


def _sc_gather(table, params):
    mesh = plsc.VectorSubcoreMesh(core_axis_name="c", subcore_axis_name="s")
    fn = functools.partial(
        pl.kernel,
        out_type=jax.ShapeDtypeStruct((NG * GSZ,), jnp.float32),
        mesh=mesh,
        scratch_types=[
            pltpu.VMEM((ROWS_PAD,), jnp.int32),
            pltpu.VMEM((ROWS_PAD,), jnp.int32),
            pltpu.VMEM((GPG, 16), jnp.int32),
            pltpu.VMEM((ROWS_PAD, CHUNK), jnp.float32),
            pltpu.VMEM((OUT_STG,), jnp.float32),
            pltpu.SemaphoreType.DMA,
            pltpu.SemaphoreType.DMA,
            pltpu.SemaphoreType.DMA,
        ],
        compiler_params=pltpu.CompilerParams(use_tc_tiling_on_sc=False),
    )(_sc_body)
    return fn(table, params)


def kernel(images, anc_bases):
    xy = anc_bases[:, :, :2]
    cen = (xy + xy) // 2.0
    cxi = cen[..., 0].astype(jnp.int32)
    cyi = cen[..., 1].astype(jnp.int32)
    selv = ((cxi & 63) >= 34).astype(jnp.int32)
    u = cxi + 32 * selv
    chunk = u >> 6
    p = u & 63
    bidx = jnp.arange(B, dtype=jnp.int32)[:, None]
    sp = (bidx * (C * 2 * TH * TCHUNKS) + selv * (TH * TCHUNKS)
          + cyi * TCHUNKS + chunk)
    zeros = jnp.zeros_like(sp)
    params = jnp.stack(
        [sp, p] + [zeros] * 14, axis=-1
    ).reshape(NG, 16)

    table = _build_table(images)
    out_flat = _sc_gather(table, params)
    rois = out_flat.reshape(B, N, C, GH, GW)
    return (rois, anc_bases[:, :, :2])


# speedup vs baseline: 574.3928x; 1.3722x over previous
import functools

import jax
import jax.numpy as jnp
from jax import lax
from jax.experimental import pallas as pl
from jax.experimental.pallas import tpu as pltpu
from jax.experimental.pallas import tpu_sc as plsc

B, C, H, W = 8, 3, 512, 512
N = 600
GH = GW = 31
NLANE = 128
NT = 5
NTILES = 32
TH = 544
TCP = 5
CHUNK = 64
NROWS_TBL = B * C * 2 * TCP * TH * 2
NUNITS = NT * B * C
YROWS = C * GH * GW
YCOLS = NT * B * NLANE

_BC_STRIDE = 2 * TCP * TH * 2


def _table_body(img_ref, out_ref):
    img = img_ref[0]
    p = jnp.pad(img, ((1, 1), (1, 1)))
    a = ((p[0:513, 0:513] + p[0:513, 1:514]) + p[1:514, 0:513]) + p[1:514, 1:514]
    s = 0.25 * a
    s0 = jnp.pad(s, ((15, 16), (15, 112)))
    s1 = jnp.pad(s, ((15, 16), (47, 80)))
    for t in range(TCP):
        out_ref[0, 0, t] = s0[:, t * 128:(t + 1) * 128]
        out_ref[0, 1, t] = s1[:, t * 128:(t + 1) * 128]


def _build_table(images):
    imgs = images.reshape(B * C, H, W)
    tab = pl.pallas_call(
        _table_body,
        grid=(B * C,),
        in_specs=[pl.BlockSpec((1, H, W), lambda i: (i, 0, 0))],
        out_specs=pl.BlockSpec((1, 2, TCP, TH, 128), lambda i: (i, 0, 0, 0, 0)),
        out_shape=jax.ShapeDtypeStruct((B * C, 2, TCP, TH, 128), jnp.float32),
    )(imgs)
    return tab.reshape(NROWS_TBL, CHUNK)


def _sc_body(table_hbm, psp_hbm, pp_hbm, out_hbm, sp_v, p_v, idx_v,
             stage_v, ostage_v, sem_g, sem_p, sem_o):
    wid = lax.axis_index("s") * 2 + lax.axis_index("c")

    def unit_body(k, _):
        u = wid + NTILES * k

        @pl.when(u < NUNITS)
        def _():
            pair = (u * 43691) >> 17
            c = u - pair * 3
            t = pair >> 3
            b = pair - (t << 3)
            col0 = t * (B * NLANE) + b * NLANE
            cterm = c * _BC_STRIDE

            cp_sp = pltpu.async_copy(psp_hbm.at[pair], sp_v, sem_p)
            cp_p = pltpu.async_copy(pp_hbm.at[pair], p_v, sem_p)
            cp_sp.wait()
            cp_p.wait()

            def i_body(i, _):
                for v in range(NLANE // 16):
                    idx_v[pl.ds(16 * v, 16)] = (
                        sp_v[pl.ds(16 * v, 16)] + (cterm + i * 2)
                    )
                pltpu.async_copy(table_hbm.at[idx_v], stage_v, sem_g).wait()

                def j_body(j, _):
                    for v in range(NLANE // 16):
                        rowv = lax.iota(jnp.int32, 16) + (16 * v)
                        colv = p_v[pl.ds(16 * v, 16)] + j
                        vec = plsc.load_gather(stage_v, [rowv, colv])
                        ostage_v[j, pl.ds(16 * v, 16)] = vec
                    return 0

                lax.fori_loop(0, GW, j_body, 0)
                r0 = (c * GH + i) * GW
                pltpu.async_copy(
                    ostage_v,
                    out_hbm.at[pl.ds(r0, GW), pl.ds(col0, NLANE)],
                    sem_o,
                ).wait()
                return 0

            lax.fori_loop(0, GH, i_body, 0)

        return 0

    lax.fori_loop(0, (NUNITS + NTILES - 1) // NTILES, unit_body, 0)


def _sc_gather(table, psp, pp):
    mesh = plsc.VectorSubcoreMesh(core_axis_name="c", subcore_axis_name="s")
    fn = functools.partial(
        pl.kernel,
        out_type=jax.ShapeDtypeStruct((YROWS, YCOLS), jnp.float32),
        mesh=mesh,
        scratch_types=[
            pltpu.VMEM((NLANE,), jnp.int32),
            pltpu.VMEM((NLANE,), jnp.int32),
            pltpu.VMEM((NLANE,), jnp.int32),
            pltpu.VMEM((NLANE, CHUNK), jnp.float32),
            pltpu.VMEM((GW, NLANE), jnp.float32),
            pltpu.SemaphoreType.DMA,
            pltpu.SemaphoreType.DMA,
            pltpu.SemaphoreType.DMA,
        ],
        compiler_params=pltpu.CompilerParams(
            use_tc_tiling_on_sc=False, needs_layout_passes=False
        ),
    )(_sc_body)
    return fn(table, psp, pp)


def kernel(images, anc_bases):
    xy = anc_bases[:, :, :2]
    cen = (xy + xy) // 2.0
    cxi = cen[..., 0].astype(jnp.int32)
    cyi = cen[..., 1].astype(jnp.int32)
    selv = ((cxi & 63) >= 34).astype(jnp.int32)
    u = cxi + 32 * selv
    c64 = u >> 6
    p = u & 63
    tch = c64 >> 1
    h = c64 & 1
    bidx = jnp.arange(B, dtype=jnp.int32)[:, None]
    sp = (bidx * (C * _BC_STRIDE) + selv * (TCP * TH * 2)
          + tch * (TH * 2) + cyi * 2 + h)
    psp = jnp.pad(sp, ((0, 0), (0, NT * NLANE - N))).reshape(
        B, NT, NLANE).transpose(1, 0, 2).reshape(NT * B, NLANE)
    pp = jnp.pad(p, ((0, 0), (0, NT * NLANE - N))).reshape(
        B, NT, NLANE).transpose(1, 0, 2).reshape(NT * B, NLANE)

    table = _build_table(images)
    out2d = _sc_gather(table, psp, pp)
    y = out2d.reshape(C, GH, GW, NT, B, NLANE)
    rois = y.transpose(4, 3, 5, 0, 1, 2).reshape(B, NT * NLANE, C, GH, GW)[:, :N]
    return (rois, anc_bases[:, :, :2])


# speedup vs baseline: 1452.9311x; 2.5295x over previous
import functools

import jax
import jax.numpy as jnp
from jax import lax
from jax.experimental import pallas as pl
from jax.experimental.pallas import tpu as pltpu
from jax.experimental.pallas import tpu_sc as plsc

B, C, H, W = 8, 3, 512, 512
N = 600
GH = GW = 31
NLANE = 128
NT = 5
NTILES = 32
TH = 544
TCP = 5
CHUNK = 64
NROWS_TBL = B * C * 2 * TCP * TH * 2
NUNITS = NT * B * C
YROWS = C * GH * GW
YCOLS = NT * B * NLANE

_BC_STRIDE = 2 * TCP * TH * 2


def _table_body(img_ref, out_ref):
    img = img_ref[0]
    p = jnp.pad(img, ((1, 1), (1, 1)))
    a = ((p[0:513, 0:513] + p[0:513, 1:514]) + p[1:514, 0:513]) + p[1:514, 1:514]
    s = 0.25 * a
    s0 = jnp.pad(s, ((15, 16), (15, 112)))
    s1 = jnp.pad(s, ((15, 16), (47, 80)))
    for t in range(TCP):
        out_ref[0, 0, t] = s0[:, t * 128:(t + 1) * 128]
        out_ref[0, 1, t] = s1[:, t * 128:(t + 1) * 128]


def _build_table(images):
    imgs = images.reshape(B * C, H, W)
    tab = pl.pallas_call(
        _table_body,
        grid=(B * C,),
        in_specs=[pl.BlockSpec((1, H, W), lambda i: (i, 0, 0))],
        out_specs=pl.BlockSpec((1, 2, TCP, TH, 128), lambda i: (i, 0, 0, 0, 0)),
        out_shape=jax.ShapeDtypeStruct((B * C, 2, TCP, TH, 128), jnp.float32),
    )(imgs)
    return tab.reshape(NROWS_TBL, CHUNK)


def _sc_body(table_hbm, psp_hbm, pp_hbm, out_hbm, sp_v, p_v, idx0_v, idx1_v,
             idx2_v, stage0_v, stage1_v, stage2_v, ost0_v, ost1_v, ost2_v,
             sem_g0, sem_g1, sem_g2, sem_o0, sem_o1, sem_o2, sem_p):
    wid = lax.axis_index("s") * 2 + lax.axis_index("c")
    idx_b = (idx0_v, idx1_v, idx2_v)
    stage_b = (stage0_v, stage1_v, stage2_v)
    ost_b = (ost0_v, ost1_v, ost2_v)
    sem_gb = (sem_g0, sem_g1, sem_g2)
    sem_ob = (sem_o0, sem_o1, sem_o2)

    def unit_body(k, _):
        u = wid + NTILES * k

        @pl.when(u < NUNITS)
        def _():
            pair = (u * 43691) >> 17
            c = u - pair * 3
            t = pair >> 3
            b = pair - (t << 3)
            col0 = t * (B * NLANE) + b * NLANE
            cterm = c * _BC_STRIDE

            cp_sp = pltpu.async_copy(psp_hbm.at[pair], sp_v, sem_p)
            cp_p = pltpu.async_copy(pp_hbm.at[pair], p_v, sem_p)
            cp_sp.wait()
            cp_p.wait()

            def build_idx(i, par):
                for v in range(NLANE // 16):
                    idx_b[par][pl.ds(16 * v, 16)] = (
                        sp_v[pl.ds(16 * v, 16)] + (cterm + i * 2)
                    )

            def start_gather(par):
                return pltpu.async_copy(
                    table_hbm.at[idx_b[par]], stage_b[par], sem_gb[par]
                )

            def extract(par):
                @plsc.parallel_loop(0, GW, unroll=8)
                def j_body(j):
                    for v in range(NLANE // 16):
                        rowv = lax.iota(jnp.int32, 16) + (16 * v)
                        colv = p_v[pl.ds(16 * v, 16)] + j
                        vec = plsc.load_gather(stage_b[par], [rowv, colv])
                        ost_b[par][j, pl.ds(16 * v, 16)] = vec

            def start_out(i, par):
                r0 = (c * GH + i) * GW
                return pltpu.async_copy(
                    ost_b[par],
                    out_hbm.at[pl.ds(r0, GW), pl.ds(col0, NLANE)],
                    sem_ob[par],
                )

            def wait_gather(par):
                pltpu.make_async_copy(
                    table_hbm.at[idx_b[par]], stage_b[par], sem_gb[par]
                ).wait()

            def wait_out(i, par):
                r0 = (c * GH + i) * GW
                pltpu.make_async_copy(
                    ost_b[par],
                    out_hbm.at[pl.ds(r0, GW), pl.ds(col0, NLANE)],
                    sem_ob[par],
                ).wait()

            build_idx(0, 0)
            start_gather(0)
            build_idx(1, 1)
            start_gather(1)

            def triple_body(i3, _):
                i0 = 3 * i3
                for s in range(3):
                    i = i0 + s
                    q = s
                    qn = (s + 2) % 3

                    @pl.when(i + 2 <= GH - 1)
                    def _():
                        build_idx(i + 2, qn)
                        start_gather(qn)

                    wait_gather(q)

                    @pl.when(i3 >= 1)
                    def _():
                        wait_out(i - 3, q)

                    extract(q)
                    start_out(i, q)
                return 0

            lax.fori_loop(0, GH // 3, triple_body, 0)
            wait_gather(0)
            wait_out(GH - 4, 0)
            extract(0)
            start_out(GH - 1, 0)
            wait_out(GH - 3, 1)
            wait_out(GH - 2, 2)
            wait_out(GH - 1, 0)

        return 0

    lax.fori_loop(0, (NUNITS + NTILES - 1) // NTILES, unit_body, 0)


def _sc_gather(table, psp, pp):
    mesh = plsc.VectorSubcoreMesh(core_axis_name="c", subcore_axis_name="s")
    fn = functools.partial(
        pl.kernel,
        out_type=jax.ShapeDtypeStruct((YROWS, YCOLS), jnp.float32),
        mesh=mesh,
        scratch_types=[
            pltpu.VMEM((NLANE,), jnp.int32),
            pltpu.VMEM((NLANE,), jnp.int32),
            pltpu.VMEM((NLANE,), jnp.int32),
            pltpu.VMEM((NLANE,), jnp.int32),
            pltpu.VMEM((NLANE,), jnp.int32),
            pltpu.VMEM((NLANE, CHUNK), jnp.float32),
            pltpu.VMEM((NLANE, CHUNK), jnp.float32),
            pltpu.VMEM((NLANE, CHUNK), jnp.float32),
            pltpu.VMEM((GW, NLANE), jnp.float32),
            pltpu.VMEM((GW, NLANE), jnp.float32),
            pltpu.VMEM((GW, NLANE), jnp.float32),
            pltpu.SemaphoreType.DMA,
            pltpu.SemaphoreType.DMA,
            pltpu.SemaphoreType.DMA,
            pltpu.SemaphoreType.DMA,
            pltpu.SemaphoreType.DMA,
            pltpu.SemaphoreType.DMA,
            pltpu.SemaphoreType.DMA,
        ],
        compiler_params=pltpu.CompilerParams(
            use_tc_tiling_on_sc=False, needs_layout_passes=False
        ),
    )(_sc_body)
    return fn(table, psp, pp)


def kernel(images, anc_bases):
    xy = anc_bases[:, :, :2]
    cen = (xy + xy) // 2.0
    cxi = cen[..., 0].astype(jnp.int32)
    cyi = cen[..., 1].astype(jnp.int32)
    selv = ((cxi & 63) >= 34).astype(jnp.int32)
    u = cxi + 32 * selv
    c64 = u >> 6
    p = u & 63
    tch = c64 >> 1
    h = c64 & 1
    bidx = jnp.arange(B, dtype=jnp.int32)[:, None]
    sp = (bidx * (C * _BC_STRIDE) + selv * (TCP * TH * 2)
          + tch * (TH * 2) + cyi * 2 + h)
    psp = jnp.pad(sp, ((0, 0), (0, NT * NLANE - N))).reshape(
        B, NT, NLANE).transpose(1, 0, 2).reshape(NT * B, NLANE)
    pp = jnp.pad(p, ((0, 0), (0, NT * NLANE - N))).reshape(
        B, NT, NLANE).transpose(1, 0, 2).reshape(NT * B, NLANE)

    table = _build_table(images)
    out2d = _sc_gather(table, psp, pp)
    y = out2d.reshape(C, GH, GW, NT, B, NLANE)
    rois = y.transpose(4, 3, 5, 0, 1, 2).reshape(B, NT * NLANE, C, GH, GW)[:, :N]
    return (rois, anc_bases[:, :, :2])


# speedup vs baseline: 1595.2015x; 1.0979x over previous
import functools

import jax
import jax.numpy as jnp
from jax import lax
from jax.experimental import pallas as pl
from jax.experimental.pallas import tpu as pltpu
from jax.experimental.pallas import tpu_sc as plsc

B, C, H, W = 8, 3, 512, 512
N = 600
GH = GW = 31
NLANE = 128
NT = 5
NTILES = 32
TH = 544
TCP = 5
CHUNK = 64
NROWS_TBL = B * C * 2 * TCP * TH * 2
NUNITS = NT * B * C
YROWS = C * GH * GW
YCOLS = NT * B * NLANE

_BC_STRIDE = 2 * TCP * TH * 2


def _table_body(img_ref, out_ref):
    img = img_ref[0]
    p = jnp.pad(img, ((1, 1), (1, 1)))
    a = ((p[0:513, 0:513] + p[0:513, 1:514]) + p[1:514, 0:513]) + p[1:514, 1:514]
    s = 0.25 * a
    s0 = jnp.pad(s, ((15, 16), (15, 112)))
    s1 = jnp.pad(s, ((15, 16), (47, 80)))
    for t in range(TCP):
        out_ref[0, 0, t] = s0[:, t * 128:(t + 1) * 128]
        out_ref[0, 1, t] = s1[:, t * 128:(t + 1) * 128]


def _build_table(images):
    imgs = images.reshape(B * C, H, W)
    tab = pl.pallas_call(
        _table_body,
        grid=(B * C,),
        in_specs=[pl.BlockSpec((1, H, W), lambda i: (i, 0, 0))],
        out_specs=pl.BlockSpec((1, 2, TCP, TH, 128), lambda i: (i, 0, 0, 0, 0)),
        out_shape=jax.ShapeDtypeStruct((B * C, 2, TCP, TH, 128), jnp.float32),
    )(imgs)
    return tab.reshape(NROWS_TBL, CHUNK)


def _sc_body(table_hbm, psp_hbm, pp_hbm, out_hbm, sp_v, p_v, idx0_v, idx1_v,
             idx2_v, stage0_v, stage1_v, stage2_v, ost_v,
             sem_g0, sem_g1, sem_g2, sem_o, sem_p):
    wid = lax.axis_index("s") * 2 + lax.axis_index("c")
    idx_b = (idx0_v, idx1_v, idx2_v)
    stage_b = (stage0_v, stage1_v, stage2_v)
    sem_gb = (sem_g0, sem_g1, sem_g2)

    def unit_body(k, _):
        u = wid + NTILES * k

        @pl.when(u < NUNITS)
        def _():
            pair = (u * 43691) >> 17
            c = u - pair * 3
            t = pair >> 3
            b = pair - (t << 3)
            col0 = t * (B * NLANE) + b * NLANE
            cterm = c * _BC_STRIDE

            cp_sp = pltpu.async_copy(psp_hbm.at[pair], sp_v, sem_p)
            cp_p = pltpu.async_copy(pp_hbm.at[pair], p_v, sem_p)
            cp_sp.wait()
            cp_p.wait()

            def build_idx(i, par):
                for v in range(NLANE // 16):
                    idx_b[par][pl.ds(16 * v, 16)] = (
                        sp_v[pl.ds(16 * v, 16)] + (cterm + i * 2)
                    )

            def start_gather(par):
                return pltpu.async_copy(
                    table_hbm.at[idx_b[par]], stage_b[par], sem_gb[par]
                )

            def extract(par, s):
                @plsc.parallel_loop(0, GW, unroll=4)
                def j_body(j):
                    for v in range(NLANE // 16):
                        rowv = lax.iota(jnp.int32, 16) + (16 * v)
                        colv = p_v[pl.ds(16 * v, 16)] + j
                        vec = plsc.load_gather(stage_b[par], [rowv, colv])
                        ost_v[s * GW + j, pl.ds(16 * v, 16)] = vec

            def out_copy(i0, nrows):
                r0 = (c * GH + i0) * GW
                return pltpu.make_async_copy(
                    ost_v.at[pl.ds(0, nrows)],
                    out_hbm.at[pl.ds(r0, nrows), pl.ds(col0, NLANE)],
                    sem_o,
                )

            def wait_gather(par):
                pltpu.make_async_copy(
                    table_hbm.at[idx_b[par]], stage_b[par], sem_gb[par]
                ).wait()

            build_idx(0, 0)
            start_gather(0)
            build_idx(1, 1)
            start_gather(1)

            def triple_body(i3, _):
                i0 = 3 * i3

                @pl.when(i3 >= 1)
                def _():
                    out_copy(i0 - 3, 3 * GW).wait()

                for s in range(3):
                    i = i0 + s
                    q = s
                    qn = (s + 2) % 3

                    @pl.when(i + 2 <= GH - 1)
                    def _():
                        build_idx(i + 2, qn)
                        start_gather(qn)

                    wait_gather(q)
                    extract(q, s)
                out_copy(i0, 3 * GW).start()
                return 0

            lax.fori_loop(0, GH // 3, triple_body, 0)
            wait_gather(0)
            out_copy(GH - 4, 3 * GW).wait()
            extract(0, 0)
            out_copy(GH - 1, GW).start()
            out_copy(GH - 1, GW).wait()

        return 0

    lax.fori_loop(0, (NUNITS + NTILES - 1) // NTILES, unit_body, 0)


def _sc_gather(table, psp, pp):
    mesh = plsc.VectorSubcoreMesh(core_axis_name="c", subcore_axis_name="s")
    fn = functools.partial(
        pl.kernel,
        out_type=jax.ShapeDtypeStruct((YROWS, YCOLS), jnp.float32),
        mesh=mesh,
        scratch_types=[
            pltpu.VMEM((NLANE,), jnp.int32),
            pltpu.VMEM((NLANE,), jnp.int32),
            pltpu.VMEM((NLANE,), jnp.int32),
            pltpu.VMEM((NLANE,), jnp.int32),
            pltpu.VMEM((NLANE,), jnp.int32),
            pltpu.VMEM((NLANE, CHUNK), jnp.float32),
            pltpu.VMEM((NLANE, CHUNK), jnp.float32),
            pltpu.VMEM((NLANE, CHUNK), jnp.float32),
            pltpu.VMEM((3 * GW, NLANE), jnp.float32),
            pltpu.SemaphoreType.DMA,
            pltpu.SemaphoreType.DMA,
            pltpu.SemaphoreType.DMA,
            pltpu.SemaphoreType.DMA,
            pltpu.SemaphoreType.DMA,
        ],
        compiler_params=pltpu.CompilerParams(
            use_tc_tiling_on_sc=False, needs_layout_passes=False
        ),
    )(_sc_body)
    return fn(table, psp, pp)


def kernel(images, anc_bases):
    xy = anc_bases[:, :, :2]
    cen = (xy + xy) // 2.0
    cxi = cen[..., 0].astype(jnp.int32)
    cyi = cen[..., 1].astype(jnp.int32)
    selv = ((cxi & 63) >= 34).astype(jnp.int32)
    u = cxi + 32 * selv
    c64 = u >> 6
    p = u & 63
    tch = c64 >> 1
    h = c64 & 1
    bidx = jnp.arange(B, dtype=jnp.int32)[:, None]
    sp = (bidx * (C * _BC_STRIDE) + selv * (TCP * TH * 2)
          + tch * (TH * 2) + cyi * 2 + h)
    psp = jnp.pad(sp, ((0, 0), (0, NT * NLANE - N))).reshape(
        B, NT, NLANE).transpose(1, 0, 2).reshape(NT * B, NLANE)
    pp = jnp.pad(p, ((0, 0), (0, NT * NLANE - N))).reshape(
        B, NT, NLANE).transpose(1, 0, 2).reshape(NT * B, NLANE)

    table = _build_table(images)
    out2d = _sc_gather(table, psp, pp)
    y = out2d.reshape(C, GH, GW, NT, B, NLANE)
    rois = y.transpose(4, 3, 5, 0, 1, 2).reshape(B, NT * NLANE, C, GH, GW)[:, :N]
    return (rois, anc_bases[:, :, :2])


# speedup vs baseline: 1742.2861x; 1.0922x over previous
import functools

import jax
import jax.numpy as jnp
from jax import lax
from jax.experimental import pallas as pl
from jax.experimental.pallas import tpu as pltpu
from jax.experimental.pallas import tpu_sc as plsc

B, C, H, W = 8, 3, 512, 512
N = 600
GH = GW = 31
NLANE = 128
NT = 5
NTILES = 32
TH = 544
TCP = 5
CHUNK = 64
NROWS_TBL = B * C * 2 * TCP * TH * 2
NUNITS = NT * B * C
YROWS = C * GH * GW
YCOLS = NT * B * NLANE

_BC_STRIDE = 2 * TCP * TH * 2


def _table_body(img_ref, out_ref):
    img = img_ref[0]
    p = jnp.pad(img, ((1, 1), (1, 1)))
    a = ((p[0:513, 0:513] + p[0:513, 1:514]) + p[1:514, 0:513]) + p[1:514, 1:514]
    s = 0.25 * a
    s0 = jnp.pad(s, ((15, 16), (15, 112)))
    s1 = jnp.pad(s, ((15, 16), (47, 80)))
    for t in range(TCP):
        out_ref[0, 0, t] = s0[:, t * 128:(t + 1) * 128]
        out_ref[0, 1, t] = s1[:, t * 128:(t + 1) * 128]


def _build_table(images):
    imgs = images.reshape(B * C, H, W)
    tab = pl.pallas_call(
        _table_body,
        grid=(B * C,),
        in_specs=[pl.BlockSpec((1, H, W), lambda i: (i, 0, 0))],
        out_specs=pl.BlockSpec((1, 2, TCP, TH, 128), lambda i: (i, 0, 0, 0, 0)),
        out_shape=jax.ShapeDtypeStruct((B * C, 2, TCP, TH, 128), jnp.float32),
    )(imgs)
    return tab.reshape(NROWS_TBL, CHUNK)


def _sc_body(table_hbm, psp_hbm, pp_hbm, out_hbm, sp_v, p_v, idx0_v, idx1_v,
             idx2_v, stage0_v, stage1_v, stage2_v, ost0_v, ost1_v, ost2_v,
             sem_g0, sem_g1, sem_g2, sem_o0, sem_o1, sem_o2, sem_p):
    wid = lax.axis_index("s") * 2 + lax.axis_index("c")
    idx_b = (idx0_v, idx1_v, idx2_v)
    stage_b = (stage0_v, stage1_v, stage2_v)
    ost_b = (ost0_v, ost1_v, ost2_v)
    sem_gb = (sem_g0, sem_g1, sem_g2)
    sem_ob = (sem_o0, sem_o1, sem_o2)

    def unit_body(k, _):
        u = wid + NTILES * k

        @pl.when(u < NUNITS)
        def _():
            pair = (u * 43691) >> 17
            c = u - pair * 3
            t = pair >> 3
            b = pair - (t << 3)
            col0 = t * (B * NLANE) + b * NLANE
            cterm = c * _BC_STRIDE

            cp_sp = pltpu.async_copy(psp_hbm.at[pair], sp_v, sem_p)
            cp_p = pltpu.async_copy(pp_hbm.at[pair], p_v, sem_p)
            cp_sp.wait()
            cp_p.wait()

            def build_idx(i, par):
                for v in range(NLANE // 16):
                    idx_b[par][pl.ds(16 * v, 16)] = (
                        sp_v[pl.ds(16 * v, 16)] + (cterm + i * 2)
                    )

            def start_gather(par):
                return pltpu.async_copy(
                    table_hbm.at[idx_b[par]], stage_b[par], sem_gb[par]
                )

            def extract(par):
                @plsc.parallel_loop(0, GW, unroll=4)
                def j_body(j):
                    for v in range(NLANE // 16):
                        rowv = lax.iota(jnp.int32, 16) + (16 * v)
                        colv = p_v[pl.ds(16 * v, 16)] + j
                        vec = plsc.load_gather(stage_b[par], [rowv, colv])
                        ost_b[par][j, pl.ds(16 * v, 16)] = vec

            def start_out(i, par):
                r0 = (c * GH + i) * GW
                return pltpu.async_copy(
                    ost_b[par],
                    out_hbm.at[pl.ds(r0, GW), pl.ds(col0, NLANE)],
                    sem_ob[par],
                )

            def wait_gather(par):
                pltpu.make_async_copy(
                    table_hbm.at[idx_b[par]], stage_b[par], sem_gb[par]
                ).wait()

            def wait_out(i, par):
                r0 = (c * GH + i) * GW
                pltpu.make_async_copy(
                    ost_b[par],
                    out_hbm.at[pl.ds(r0, GW), pl.ds(col0, NLANE)],
                    sem_ob[par],
                ).wait()

            build_idx(0, 0)
            start_gather(0)
            build_idx(1, 1)
            start_gather(1)

            def triple_body(i3, _):
                i0 = 3 * i3
                for s in range(3):
                    i = i0 + s
                    q = s
                    qn = (s + 2) % 3

                    @pl.when(i + 2 <= GH - 1)
                    def _():
                        build_idx(i + 2, qn)
                        start_gather(qn)

                    wait_gather(q)

                    @pl.when(i3 >= 1)
                    def _():
                        wait_out(i - 3, q)

                    extract(q)
                    start_out(i, q)
                return 0

            lax.fori_loop(0, GH // 3, triple_body, 0)
            wait_gather(0)
            wait_out(GH - 4, 0)
            extract(0)
            start_out(GH - 1, 0)
            wait_out(GH - 3, 1)
            wait_out(GH - 2, 2)
            wait_out(GH - 1, 0)

        return 0

    lax.fori_loop(0, (NUNITS + NTILES - 1) // NTILES, unit_body, 0)


def _sc_gather(table, psp, pp):
    mesh = plsc.VectorSubcoreMesh(core_axis_name="c", subcore_axis_name="s")
    fn = functools.partial(
        pl.kernel,
        out_type=jax.ShapeDtypeStruct((YROWS, YCOLS), jnp.float32),
        mesh=mesh,
        scratch_types=[
            pltpu.VMEM((NLANE,), jnp.int32),
            pltpu.VMEM((NLANE,), jnp.int32),
            pltpu.VMEM((NLANE,), jnp.int32),
            pltpu.VMEM((NLANE,), jnp.int32),
            pltpu.VMEM((NLANE,), jnp.int32),
            pltpu.VMEM((NLANE, CHUNK), jnp.float32),
            pltpu.VMEM((NLANE, CHUNK), jnp.float32),
            pltpu.VMEM((NLANE, CHUNK), jnp.float32),
            pltpu.VMEM((GW, NLANE), jnp.float32),
            pltpu.VMEM((GW, NLANE), jnp.float32),
            pltpu.VMEM((GW, NLANE), jnp.float32),
            pltpu.SemaphoreType.DMA,
            pltpu.SemaphoreType.DMA,
            pltpu.SemaphoreType.DMA,
            pltpu.SemaphoreType.DMA,
            pltpu.SemaphoreType.DMA,
            pltpu.SemaphoreType.DMA,
            pltpu.SemaphoreType.DMA,
        ],
        compiler_params=pltpu.CompilerParams(
            use_tc_tiling_on_sc=False, needs_layout_passes=False
        ),
    )(_sc_body)
    return fn(table, psp, pp)


def kernel(images, anc_bases):
    xy = anc_bases[:, :, :2]
    cen = (xy + xy) // 2.0
    cxi = cen[..., 0].astype(jnp.int32)
    cyi = cen[..., 1].astype(jnp.int32)
    selv = ((cxi & 63) >= 34).astype(jnp.int32)
    u = cxi + 32 * selv
    c64 = u >> 6
    p = u & 63
    tch = c64 >> 1
    h = c64 & 1
    bidx = jnp.arange(B, dtype=jnp.int32)[:, None]
    sp = (bidx * (C * _BC_STRIDE) + selv * (TCP * TH * 2)
          + tch * (TH * 2) + cyi * 2 + h)
    psp = jnp.pad(sp, ((0, 0), (0, NT * NLANE - N))).reshape(
        B, NT, NLANE).transpose(1, 0, 2).reshape(NT * B, NLANE)
    pp = jnp.pad(p, ((0, 0), (0, NT * NLANE - N))).reshape(
        B, NT, NLANE).transpose(1, 0, 2).reshape(NT * B, NLANE)

    table = _build_table(images)
    out2d = _sc_gather(table, psp, pp)
    y = out2d.reshape(C, GH, GW, NT, B, NLANE)
    rois = y.transpose(4, 3, 5, 0, 1, 2).reshape(B, NT * NLANE, C, GH, GW)[:, :N]
    return (rois, anc_bases[:, :, :2])
